# baseline (speedup 1.0000x reference)
import functools

import jax
import jax.numpy as jnp
from jax.experimental import pallas as pl

NUM_GRAPHS = 16


def _leaky_relu(x, slope=0.01):
    return jnp.where(x >= 0, x, slope * x)


def _pool_kernel(ohT_ref, x_ref, out_ref):
    k = pl.program_id(0)
    acc = jnp.dot(ohT_ref[...], x_ref[...], preferred_element_type=jnp.float32)

    @pl.when(k == 0)
    def _init():
        out_ref[...] = acc

    @pl.when(k > 0)
    def _acc():
        out_ref[...] += acc


def _pool(x, batch):
    n, c = x.shape
    blk = 1024
    n_pad = ((n + blk - 1) // blk) * blk
    x_pad = jnp.pad(x, ((0, n_pad - n), (0, 0)))
    batch_pad = jnp.pad(batch.astype(jnp.int32), (0, n_pad - n),
                        constant_values=NUM_GRAPHS)
    ohT = (batch_pad[None, :] == jnp.arange(NUM_GRAPHS, dtype=jnp.int32)[:, None])
    ohT = ohT.astype(jnp.float32)
    sums = pl.pallas_call(
        _pool_kernel,
        grid=(n_pad // blk,),
        in_specs=[
            pl.BlockSpec((NUM_GRAPHS, blk), lambda k: (0, k)),
            pl.BlockSpec((blk, c), lambda k: (k, 0)),
        ],
        out_specs=pl.BlockSpec((NUM_GRAPHS, c), lambda k: (0, 0)),
        out_shape=jax.ShapeDtypeStruct((NUM_GRAPHS, c), jnp.float32),
    )(ohT, x_pad)
    counts = jnp.sum(ohT, axis=1)
    counts = jnp.maximum(counts, 1.0)
    return sums / counts[:, None]


def _conv_simplified(x, src, dst, edge_attr, p):
    n = x.shape[0]
    q = x @ p["Wq"] + p["bq"]
    k = x @ p["Wk"] + p["bk"]
    v = x @ p["Wv"] + p["bv"]
    qt = q @ p["We"].T
    C = q.shape[-1]
    alpha = (jnp.sum(q[dst] * k[src], axis=-1)
             + jnp.sum(qt[dst] * edge_attr, axis=-1)) / jnp.sqrt(jnp.float32(C))
    m = jax.ops.segment_max(alpha, dst, num_segments=n)
    m = jnp.where(jnp.isfinite(m), m, 0.0)
    ex = jnp.exp(alpha - m[dst])
    denom = jax.ops.segment_sum(ex, dst, num_segments=n)
    a = ex / (denom[dst] + 1e-16)
    out_v = jax.ops.segment_sum(v[src] * a[:, None], dst, num_segments=n)
    u = jax.ops.segment_sum(edge_attr * a[:, None], dst, num_segments=n)
    out = out_v + u @ p["We"] + x @ p["Ws"] + p["bs"]
    return out


def kernel(seq, edge_index, node_s, esm_emb, edge_s, batch, params):
    x = jnp.concatenate([seq, node_s, esm_emb], axis=-1)
    x = x @ params["proj_node"]["W"] + params["proj_node"]["b"]
    edge_attr = edge_s @ params["proj_edge"]["W"] + params["proj_edge"]["b"]
    src = edge_index[0].astype(jnp.int32)
    dst = edge_index[1].astype(jnp.int32)
    for lp in params["layers"]:
        x = _conv_simplified(x, src, dst, edge_attr, lp)
        x = _leaky_relu(x)
    return _pool(x, batch)


# speedup vs baseline: 2.5897x; 2.5897x over previous
import functools

import jax
import jax.numpy as jnp
import numpy as np
from jax import lax
from jax.experimental import pallas as pl
from jax.experimental.pallas import tpu as pltpu
from jax.experimental.pallas import tpu_sc as plsc

N_GRAPHS = 16
NC, NS = 2, 16
NW = NC * NS
NOLAYOUT = pltpu.CompilerParams(needs_layout_passes=False)
N_PAD = 10240
EPS = 1e-16


def _mm_kernel(a_ref, b_ref, bias_ref, o_ref):
    o_ref[...] = (jnp.dot(a_ref[...], b_ref[...],
                          preferred_element_type=jnp.float32) + bias_ref[...])


def _mm(A, B, bias, blk=1024):
    M, K = A.shape
    N = B.shape[1]
    return pl.pallas_call(
        _mm_kernel,
        grid=(M // blk,),
        in_specs=[
            pl.BlockSpec((blk, K), lambda i: (i, 0)),
            pl.BlockSpec((K, N), lambda i: (0, 0)),
            pl.BlockSpec((1, N), lambda i: (0, 0)),
        ],
        out_specs=pl.BlockSpec((blk, N), lambda i: (i, 0)),
        out_shape=jax.ShapeDtypeStruct((M, N), jnp.float32),
    )(A, B, bias.reshape(1, N))


def _proj_node_kernel(small_ref, w0_ref, esm_ref, w2_ref, bias_ref, o_ref):
    k = pl.program_id(1)

    @pl.when(k == 0)
    def _init():
        o_ref[...] = (jnp.dot(small_ref[...], w0_ref[...],
                              preferred_element_type=jnp.float32) + bias_ref[...])

    @pl.when(k > 0)
    def _acc():
        o_ref[...] += jnp.dot(esm_ref[...], w2_ref[...],
                              preferred_element_type=jnp.float32)


def _proj_node(small, w0, esm, w2, bias, blk=1024, kblk=512):
    M = small.shape[0]
    N = w0.shape[1]
    nk = esm.shape[1] // kblk
    return pl.pallas_call(
        _proj_node_kernel,
        grid=(M // blk, nk + 1),
        in_specs=[
            pl.BlockSpec((blk, small.shape[1]), lambda i, k: (i, 0)),
            pl.BlockSpec((small.shape[1], N), lambda i, k: (0, 0)),
            pl.BlockSpec((blk, kblk), lambda i, k: (i, jnp.maximum(k - 1, 0))),
            pl.BlockSpec((kblk, N), lambda i, k: (jnp.maximum(k - 1, 0), 0)),
            pl.BlockSpec((1, N), lambda i, k: (0, 0)),
        ],
        out_specs=pl.BlockSpec((blk, N), lambda i, k: (i, 0)),
        out_shape=jax.ShapeDtypeStruct((M, N), jnp.float32),
    )(small, w0, esm, w2, bias.reshape(1, N))


def _epilogue_kernel(dout, eaw, r_ref, x_ref, wcat_ref, bias_ref, o_ref):
    r = r_ref[...]
    den = lax.slice(r, (0, dout + eaw), (r.shape[0], dout + eaw + 1))
    recip = 1.0 / (den + EPS)
    rv = lax.slice(r, (0, 0), (r.shape[0], dout)) * recip
    ru = lax.slice(r, (0, dout), (r.shape[0], dout + eaw)) * recip
    a = jnp.concatenate([ru, x_ref[...]], axis=1)
    out = rv + jnp.dot(a, wcat_ref[...],
                       preferred_element_type=jnp.float32) + bias_ref[...]
    o_ref[...] = jnp.where(out >= 0, out, 0.01 * out)


def _epilogue(R, x, Wcat, bias, dout, eaw, blk=1024):
    M, W = R.shape
    kdim = Wcat.shape[0]
    return pl.pallas_call(
        functools.partial(_epilogue_kernel, dout, eaw),
        grid=(M // blk,),
        in_specs=[
            pl.BlockSpec((blk, W), lambda i: (i, 0)),
            pl.BlockSpec((blk, x.shape[1]), lambda i: (i, 0)),
            pl.BlockSpec((kdim, dout), lambda i: (0, 0)),
            pl.BlockSpec((1, dout), lambda i: (0, 0)),
        ],
        out_specs=pl.BlockSpec((blk, dout), lambda i: (i, 0)),
        out_shape=jax.ShapeDtypeStruct((M, dout), jnp.float32),
    )(R, x, Wcat, bias.reshape(1, dout))


def _pool_kernel(ohT_ref, x_ref, sum_ref, cnt_ref):
    k = pl.program_id(0)
    oh = ohT_ref[...]
    acc = jnp.dot(oh, x_ref[...], preferred_element_type=jnp.float32)
    c = jnp.sum(oh, axis=1, keepdims=True) + jnp.zeros_like(cnt_ref)

    @pl.when(k == 0)
    def _init():
        sum_ref[...] = acc
        cnt_ref[...] = c

    @pl.when(k > 0)
    def _acc():
        sum_ref[...] += acc
        cnt_ref[...] += c

    @pl.when(k == pl.num_programs(0) - 1)
    def _fin():
        sum_ref[...] = sum_ref[...] / jnp.maximum(cnt_ref[...], 1.0)


def _pool(x, ohT, blk=1024):
    M, C = x.shape
    out, _ = pl.pallas_call(
        _pool_kernel,
        grid=(M // blk,),
        in_specs=[
            pl.BlockSpec((N_GRAPHS, blk), lambda k: (0, k)),
            pl.BlockSpec((blk, C), lambda k: (k, 0)),
        ],
        out_specs=[
            pl.BlockSpec((N_GRAPHS, C), lambda k: (0, 0)),
            pl.BlockSpec((N_GRAPHS, C), lambda k: (0, 0)),
        ],
        out_shape=[
            jax.ShapeDtypeStruct((N_GRAPHS, C), jnp.float32),
            jax.ShapeDtypeStruct((N_GRAPHS, C), jnp.float32),
        ],
    )(ohT, x)
    return out


def _make_sweep(n_nodes, dout, eaw, E):
    W = dout + eaw + 16
    rsqrt_c = 1.0 / float(np.sqrt(np.float32(dout)))
    mesh = plsc.VectorSubcoreMesh(core_axis_name="c", subcore_axis_name="s")

    @functools.partial(
        pl.kernel,
        out_type=jax.ShapeDtypeStruct((n_nodes, 1, W), jnp.float32),
        mesh=mesh,
        compiler_params=NOLAYOUT,
        scratch_types=[
            pltpu.VMEM((16,), jnp.int32),
            pltpu.VMEM((E,), jnp.int32),
            pltpu.VMEM((E,), jnp.int32),
            pltpu.VMEM((E,), jnp.int32),
            pltpu.VMEM((E, dout), jnp.float32),
            pltpu.VMEM((E, dout + eaw), jnp.float32),
            pltpu.VMEM((E, dout), jnp.float32),
            pltpu.VMEM((E, eaw), jnp.float32),
            pltpu.VMEM((16, 1, W), jnp.float32),
            pltpu.SMEM((8,), jnp.int32),
            pltpu.SemaphoreType.DMA,
            pltpu.SemaphoreType.DMA,
            pltpu.SemaphoreType.DMA,
            pltpu.SemaphoreType.DMA,
        ],
    )
    def sweep(ktab, qqt, vtab, eatab, srcp, dstp, permp, meta, outR,
              metav, srcb, dstb, permb, kb, qb, vb, eab, win, st,
              s1, s2, s3, s4):
        cid = lax.axis_index("c")
        sid = lax.axis_index("s")
        w = sid * NC + cid
        pltpu.sync_copy(meta.at[pl.ds(pl.multiple_of(w * 16, 16), 16)], metav)
        mv = metav[...]
        ts0, ts1, f0, f1 = mv[0], mv[1], mv[2], mv[3]
        cbase0 = pl.multiple_of((ts0 // 8) * 8, 8)
        nch = lax.div(ts1 - cbase0 + E - 1, E)
        st[0] = f0
        st[1] = 0
        st[2] = f0
        for r in range(16):
            for j in range(W // 16):
                win[r, 0, pl.ds(j * 16, 16)] = jnp.zeros((16,), jnp.float32)

        def adv(i, carry):
            wi = st[1] + 1

            @pl.when(wi == 16)
            def _fl():
                fb = st[2]
                pltpu.sync_copy(win, outR.at[pl.ds(fb, 16)])
                for r in range(16):
                    for j in range(W // 16):
                        win[r, 0, pl.ds(j * 16, 16)] = jnp.zeros((16,), jnp.float32)
                st[2] = fb + 16
                st[1] = 0

            @pl.when(wi < 16)
            def _nf():
                st[1] = wi

            return carry

        def chunk(k, carry):
            cb = pl.multiple_of(cbase0 + k * E, 8)
            pltpu.sync_copy(srcp.at[pl.ds(cb, E)], srcb)
            pltpu.sync_copy(dstp.at[pl.ds(cb, E)], dstb)
            pltpu.sync_copy(permp.at[pl.ds(cb, E)], permb)
            c1 = pltpu.async_copy(ktab.at[srcb], kb, s1)
            c2 = pltpu.async_copy(qqt.at[dstb], qb, s2)
            c3 = pltpu.async_copy(vtab.at[srcb], vb, s3)
            c4 = pltpu.async_copy(eatab.at[permb], eab, s4)
            c1.wait()
            c2.wait()
            c3.wait()
            c4.wait()

            def edge(el, carry2):
                d16 = plsc.load_gather(dstb, [jnp.full((16,), el, jnp.int32)])
                d = d16[0]
                lax.fori_loop(0, d - st[0], adv, 0)
                st[0] = d
                acc = jnp.zeros((16,), jnp.float32)
                for j in range(dout // 16):
                    acc = acc + (kb[el, pl.ds(j * 16, 16)]
                                 * qb[el, pl.ds(j * 16, 16)])
                for j in range(eaw // 16):
                    acc = acc + (eab[el, pl.ds(j * 16, 16)]
                                 * qb[el, pl.ds(dout + j * 16, 16)])
                s = jnp.sum(acc) * rsqrt_c
                ex = jnp.exp(jnp.full((16,), s, jnp.float32))
                wi = st[1]
                for j in range(dout // 16):
                    win[wi, 0, pl.ds(j * 16, 16)] += ex * vb[el, pl.ds(j * 16, 16)]
                for j in range(eaw // 16):
                    win[wi, 0, pl.ds(dout + j * 16, 16)] += (
                        ex * eab[el, pl.ds(j * 16, 16)])
                win[wi, 0, pl.ds(dout + eaw, 16)] += ex
                return carry2

            estart = jnp.maximum(ts0 - cb, 0)
            eend = jnp.minimum(ts1 - cb, E)
            lax.fori_loop(estart, eend, edge, 0)
            return carry

        lax.fori_loop(0, nch, chunk, 0)
        lax.fori_loop(0, f1 - st[0], adv, 0)

        def prow(r, carry):
            pltpu.sync_copy(win.at[r], outR.at[st[2] + r])
            return carry

        lax.fori_loop(0, st[1], prow, 0)

    return sweep


def _build_edge_meta(src, dst, n_nodes_pad, e_tot, E):
    perm = jnp.argsort(dst)
    dst_s = dst[perm].astype(jnp.int32)
    src_s = src[perm].astype(jnp.int32)
    perm = perm.astype(jnp.int32)
    probes = jnp.arange(NW, dtype=jnp.int32) * (e_tot // NW)
    ts_w = jnp.searchsorted(dst_s, dst_s[probes], side="left").astype(jnp.int32)
    ts = jnp.concatenate([ts_w, jnp.array([e_tot], jnp.int32)])
    f_w = dst_s[ts_w]
    f_w = f_w.at[0].set(0)
    f = jnp.concatenate([f_w, jnp.array([n_nodes_pad], jnp.int32)])
    meta = jnp.zeros((NW, 16), jnp.int32)
    meta = meta.at[:, 0].set(ts[:NW])
    meta = meta.at[:, 1].set(ts[1:])
    meta = meta.at[:, 2].set(f[:NW])
    meta = meta.at[:, 3].set(f[1:])
    src_s = jnp.pad(src_s, (0, E))
    dst_s = jnp.pad(dst_s, (0, E))
    perm = jnp.pad(perm, (0, E))
    return src_s, dst_s, perm, meta.reshape(-1)


def kernel(seq, edge_index, node_s, esm_emb, edge_s, batch, params):
    n = seq.shape[0]
    e_tot = edge_index.shape[1]
    E = 64

    small = jnp.concatenate([seq, node_s], axis=-1)
    small = jnp.pad(small, ((0, N_PAD - n), (0, 128 - 39)))
    esm = jnp.pad(esm_emb, ((0, N_PAD - n), (0, 0)))
    wn = params["proj_node"]["W"]
    w0 = jnp.pad(wn[:39], ((0, 128 - 39), (0, 0)))
    x = _proj_node(small, w0, esm, wn[39:], params["proj_node"]["b"])

    e_pad = ((e_tot + 1023) // 1024) * 1024
    edge_sp = jnp.pad(edge_s, ((0, e_pad - e_tot), (0, 128 - edge_s.shape[1])))
    wep = jnp.pad(params["proj_edge"]["W"], ((0, 128 - edge_s.shape[1]), (0, 0)))
    eatab = _mm(edge_sp, wep, params["proj_edge"]["b"])

    src = edge_index[0].astype(jnp.int32)
    dst = edge_index[1].astype(jnp.int32)
    src_s, dst_s, perm, meta = _build_edge_meta(src, dst, N_PAD, e_tot, E)

    eaw = 128
    for lp in params["layers"]:
        din = lp["Wq"].shape[0]
        dout = lp["Wq"].shape[1]
        wq_ext = jnp.concatenate([lp["Wq"], lp["bq"][None, :]], axis=0)
        wq_ext = jnp.pad(wq_ext, ((0, 7), (0, 0)))
        wtl = _mm(wq_ext, lp["We"].T, jnp.zeros((eaw,), jnp.float32),
                  blk=din + 8)
        wqq = jnp.concatenate([lp["Wq"], wtl[:din]], axis=1)
        bqq = jnp.concatenate([lp["bq"], wtl[din]], axis=0)
        qqt = _mm(x, wqq, bqq)
        ktab = _mm(x, lp["Wk"], lp["bk"])
        vtab = _mm(x, lp["Wv"], lp["bv"])
        sweep = _make_sweep(N_PAD, dout, eaw, E)
        R = sweep(ktab, qqt, vtab, eatab, src_s, dst_s, perm, meta)
        R = R.reshape(N_PAD, dout + eaw + 16)
        wcat = jnp.concatenate([lp["We"], lp["Ws"]], axis=0)
        x = _epilogue(R, x, wcat, lp["bs"], dout, eaw)

    batch_pad = jnp.pad(batch.astype(jnp.int32), (0, N_PAD - n),
                        constant_values=N_GRAPHS)
    ohT = (batch_pad[None, :]
           == jnp.arange(N_GRAPHS, dtype=jnp.int32)[:, None]).astype(jnp.float32)
    return _pool(x, ohT)


# speedup vs baseline: 3.5219x; 1.3600x over previous
import functools

import jax
import jax.numpy as jnp
import numpy as np
from jax import lax
from jax.experimental import pallas as pl
from jax.experimental.pallas import tpu as pltpu
from jax.experimental.pallas import tpu_sc as plsc

N_GRAPHS = 16
NC, NS = 2, 16
NW = NC * NS
NOLAYOUT = pltpu.CompilerParams(needs_layout_passes=False)
N_PAD = 10240
EPS = 1e-16


def _mm_kernel(a_ref, b_ref, bias_ref, o_ref):
    o_ref[...] = (jnp.dot(a_ref[...], b_ref[...],
                          preferred_element_type=jnp.float32) + bias_ref[...])


def _mm(A, B, bias, blk=1024):
    M, K = A.shape
    N = B.shape[1]
    return pl.pallas_call(
        _mm_kernel,
        grid=(M // blk,),
        in_specs=[
            pl.BlockSpec((blk, K), lambda i: (i, 0)),
            pl.BlockSpec((K, N), lambda i: (0, 0)),
            pl.BlockSpec((1, N), lambda i: (0, 0)),
        ],
        out_specs=pl.BlockSpec((blk, N), lambda i: (i, 0)),
        out_shape=jax.ShapeDtypeStruct((M, N), jnp.float32),
    )(A, B, bias.reshape(1, N))


def _proj_node_kernel(small_ref, w0_ref, esm_ref, w2_ref, bias_ref, o_ref):
    k = pl.program_id(1)

    @pl.when(k == 0)
    def _init():
        o_ref[...] = (jnp.dot(small_ref[...], w0_ref[...],
                              preferred_element_type=jnp.float32) + bias_ref[...])

    @pl.when(k > 0)
    def _acc():
        o_ref[...] += jnp.dot(esm_ref[...], w2_ref[...],
                              preferred_element_type=jnp.float32)


def _proj_node(small, w0, esm, w2, bias, blk=1024, kblk=512):
    M = small.shape[0]
    N = w0.shape[1]
    nk = esm.shape[1] // kblk
    return pl.pallas_call(
        _proj_node_kernel,
        grid=(M // blk, nk + 1),
        in_specs=[
            pl.BlockSpec((blk, small.shape[1]), lambda i, k: (i, 0)),
            pl.BlockSpec((small.shape[1], N), lambda i, k: (0, 0)),
            pl.BlockSpec((blk, kblk), lambda i, k: (i, jnp.maximum(k - 1, 0))),
            pl.BlockSpec((kblk, N), lambda i, k: (jnp.maximum(k - 1, 0), 0)),
            pl.BlockSpec((1, N), lambda i, k: (0, 0)),
        ],
        out_specs=pl.BlockSpec((blk, N), lambda i, k: (i, 0)),
        out_shape=jax.ShapeDtypeStruct((M, N), jnp.float32),
    )(small, w0, esm, w2, bias.reshape(1, N))


def _epilogue_kernel(dout, eaw, r_ref, x_ref, wcat_ref, bias_ref, o_ref):
    r = r_ref[...]
    den = lax.slice(r, (0, dout + eaw), (r.shape[0], dout + eaw + 1))
    recip = 1.0 / (den + EPS)
    rv = lax.slice(r, (0, 0), (r.shape[0], dout)) * recip
    ru = lax.slice(r, (0, dout), (r.shape[0], dout + eaw)) * recip
    a = jnp.concatenate([ru, x_ref[...]], axis=1)
    out = rv + jnp.dot(a, wcat_ref[...],
                       preferred_element_type=jnp.float32) + bias_ref[...]
    o_ref[...] = jnp.where(out >= 0, out, 0.01 * out)


def _epilogue(R, x, Wcat, bias, dout, eaw, blk=1024):
    M, W = R.shape
    kdim = Wcat.shape[0]
    return pl.pallas_call(
        functools.partial(_epilogue_kernel, dout, eaw),
        grid=(M // blk,),
        in_specs=[
            pl.BlockSpec((blk, W), lambda i: (i, 0)),
            pl.BlockSpec((blk, x.shape[1]), lambda i: (i, 0)),
            pl.BlockSpec((kdim, dout), lambda i: (0, 0)),
            pl.BlockSpec((1, dout), lambda i: (0, 0)),
        ],
        out_specs=pl.BlockSpec((blk, dout), lambda i: (i, 0)),
        out_shape=jax.ShapeDtypeStruct((M, dout), jnp.float32),
    )(R, x, Wcat, bias.reshape(1, dout))


def _pool_kernel(ohT_ref, x_ref, sum_ref, cnt_ref):
    k = pl.program_id(0)
    oh = ohT_ref[...]
    acc = jnp.dot(oh, x_ref[...], preferred_element_type=jnp.float32)
    c = jnp.sum(oh, axis=1, keepdims=True) + jnp.zeros_like(cnt_ref)

    @pl.when(k == 0)
    def _init():
        sum_ref[...] = acc
        cnt_ref[...] = c

    @pl.when(k > 0)
    def _acc():
        sum_ref[...] += acc
        cnt_ref[...] += c

    @pl.when(k == pl.num_programs(0) - 1)
    def _fin():
        sum_ref[...] = sum_ref[...] / jnp.maximum(cnt_ref[...], 1.0)


def _pool(x, ohT, blk=1024):
    M, C = x.shape
    out, _ = pl.pallas_call(
        _pool_kernel,
        grid=(M // blk,),
        in_specs=[
            pl.BlockSpec((N_GRAPHS, blk), lambda k: (0, k)),
            pl.BlockSpec((blk, C), lambda k: (k, 0)),
        ],
        out_specs=[
            pl.BlockSpec((N_GRAPHS, C), lambda k: (0, 0)),
            pl.BlockSpec((N_GRAPHS, C), lambda k: (0, 0)),
        ],
        out_shape=[
            jax.ShapeDtypeStruct((N_GRAPHS, C), jnp.float32),
            jax.ShapeDtypeStruct((N_GRAPHS, C), jnp.float32),
        ],
    )(ohT, x)
    return out


def _make_sweep(n_nodes, dout, eaw, E):
    W = dout + eaw + 16
    rsqrt_c = 1.0 / float(np.sqrt(np.float32(dout)))
    mesh = plsc.VectorSubcoreMesh(core_axis_name="c", subcore_axis_name="s")

    @functools.partial(
        pl.kernel,
        out_type=jax.ShapeDtypeStruct((n_nodes, 1, W), jnp.float32),
        mesh=mesh,
        compiler_params=NOLAYOUT,
        scratch_types=[
            pltpu.VMEM((16,), jnp.int32),
            pltpu.VMEM((3, E), jnp.int32),
            pltpu.VMEM((3, E), jnp.int32),
            pltpu.VMEM((3, E), jnp.int32),
            pltpu.VMEM((2, E, dout), jnp.float32),
            pltpu.VMEM((2, E, dout + eaw), jnp.float32),
            pltpu.VMEM((2, E, dout), jnp.float32),
            pltpu.VMEM((2, E, eaw), jnp.float32),
            pltpu.VMEM((16, 1, W), jnp.float32),
            pltpu.SMEM((8,), jnp.int32),
            pltpu.SemaphoreType.DMA,
            pltpu.SemaphoreType.DMA,
            pltpu.SemaphoreType.DMA,
            pltpu.SemaphoreType.DMA,
            pltpu.SemaphoreType.DMA,
            pltpu.SemaphoreType.DMA,
            pltpu.SemaphoreType.DMA,
        ],
    )
    def sweep(ktab, qqt, vtab, eatab, srcp, dstp, permp, meta, outR,
              metav, srcb, dstb, permb, kb, qb, vb, eab, win, st,
              s1, s2, s3, s4, si1, si2, si3):
        cid = lax.axis_index("c")
        sid = lax.axis_index("s")
        w = sid * NC + cid
        pltpu.sync_copy(meta.at[pl.ds(pl.multiple_of(w * 16, 16), 16)], metav)
        mv = metav[...]
        ts0, ts1, f0, f1 = mv[0], mv[1], mv[2], mv[3]
        cbase0 = pl.multiple_of((ts0 // 8) * 8, 8)
        nch = lax.div(ts1 - cbase0 + E - 1, E)
        st[0] = f0
        st[1] = 0
        st[2] = f0
        for r in range(16):
            for j in range(W // 16):
                win[r, 0, pl.ds(j * 16, 16)] = jnp.zeros((16,), jnp.float32)

        def adv(i, carry):
            wi = st[1] + 1

            @pl.when(wi == 16)
            def _fl():
                fb = st[2]
                pltpu.sync_copy(win, outR.at[pl.ds(fb, 16)])
                for r in range(16):
                    for j in range(W // 16):
                        win[r, 0, pl.ds(j * 16, 16)] = jnp.zeros((16,), jnp.float32)
                st[2] = fb + 16
                st[1] = 0

            @pl.when(wi < 16)
            def _nf():
                st[1] = wi

            return carry

        pltpu.sync_copy(srcp.at[pl.ds(cbase0, E)], srcb.at[0])
        pltpu.sync_copy(dstp.at[pl.ds(cbase0, E)], dstb.at[0])
        pltpu.sync_copy(permp.at[pl.ds(cbase0, E)], permb.at[0])

        def piter(k, carry):
            p3 = lax.rem(k, 3)
            p2 = lax.rem(k, 2)
            c1 = pltpu.async_copy(ktab.at[srcb.at[p3]], kb.at[p2], s1)
            c2 = pltpu.async_copy(qqt.at[dstb.at[p3]], qb.at[p2], s2)
            c3 = pltpu.async_copy(vtab.at[srcb.at[p3]], vb.at[p2], s3)
            c4 = pltpu.async_copy(eatab.at[permb.at[p3]], eab.at[p2], s4)
            n3 = lax.rem(k + 1, 3)
            cbn = pl.multiple_of(cbase0 + (k + 1) * E, 8)
            i1 = pltpu.async_copy(srcp.at[pl.ds(cbn, E)], srcb.at[n3], si1)
            i2 = pltpu.async_copy(dstp.at[pl.ds(cbn, E)], dstb.at[n3], si2)
            i3 = pltpu.async_copy(permp.at[pl.ds(cbn, E)], permb.at[n3], si3)

            @pl.when(k > 0)
            def _compute():
                kc = k - 1
                q3 = lax.rem(kc, 3)
                q2 = lax.rem(kc, 2)
                cb = pl.multiple_of(cbase0 + kc * E, 8)

                def edge(el, carry2):
                    d16 = plsc.load_gather(
                        dstb, [jnp.full((16,), q3, jnp.int32),
                               jnp.full((16,), el, jnp.int32)])
                    d = d16[0]
                    lax.fori_loop(0, d - st[0], adv, 0)
                    st[0] = d
                    acc = jnp.zeros((16,), jnp.float32)
                    for j in range(dout // 16):
                        acc = acc + (kb[q2, el, pl.ds(j * 16, 16)]
                                     * qb[q2, el, pl.ds(j * 16, 16)])
                    for j in range(eaw // 16):
                        acc = acc + (eab[q2, el, pl.ds(j * 16, 16)]
                                     * qb[q2, el, pl.ds(dout + j * 16, 16)])
                    s = jnp.sum(acc) * rsqrt_c
                    ex = jnp.exp(jnp.full((16,), s, jnp.float32))
                    wi = st[1]
                    for j in range(dout // 16):
                        win[wi, 0, pl.ds(j * 16, 16)] += (
                            ex * vb[q2, el, pl.ds(j * 16, 16)])
                    for j in range(eaw // 16):
                        win[wi, 0, pl.ds(dout + j * 16, 16)] += (
                            ex * eab[q2, el, pl.ds(j * 16, 16)])
                    win[wi, 0, pl.ds(dout + eaw, 16)] += ex
                    return carry2

                estart = jnp.maximum(ts0 - cb, 0)
                eend = jnp.minimum(ts1 - cb, E)
                lax.fori_loop(estart, eend, edge, 0)

            c1.wait()
            c2.wait()
            c3.wait()
            c4.wait()
            i1.wait()
            i2.wait()
            i3.wait()
            return carry

        lax.fori_loop(0, nch + 1, piter, 0)
        lax.fori_loop(0, f1 - st[0], adv, 0)

        def prow(r, carry):
            pltpu.sync_copy(win.at[r], outR.at[st[2] + r])
            return carry

        lax.fori_loop(0, st[1], prow, 0)

    return sweep


def _build_edge_meta(src, dst, n_nodes_pad, e_tot):
    perm = jnp.argsort(dst)
    dst_s = dst[perm].astype(jnp.int32)
    src_s = src[perm].astype(jnp.int32)
    perm = perm.astype(jnp.int32)
    probes = jnp.arange(NW, dtype=jnp.int32) * (e_tot // NW)
    ts_w = jnp.searchsorted(dst_s, dst_s[probes], side="left").astype(jnp.int32)
    ts = jnp.concatenate([ts_w, jnp.array([e_tot], jnp.int32)])
    f_w = dst_s[ts_w]
    f_w = f_w.at[0].set(0)
    f = jnp.concatenate([f_w, jnp.array([n_nodes_pad], jnp.int32)])
    meta = jnp.zeros((NW, 16), jnp.int32)
    meta = meta.at[:, 0].set(ts[:NW])
    meta = meta.at[:, 1].set(ts[1:])
    meta = meta.at[:, 2].set(f[:NW])
    meta = meta.at[:, 3].set(f[1:])
    src_s = jnp.pad(src_s, (0, 512))
    dst_s = jnp.pad(dst_s, (0, 512))
    perm = jnp.pad(perm, (0, 512))
    return src_s, dst_s, perm, meta.reshape(-1)


def kernel(seq, edge_index, node_s, esm_emb, edge_s, batch, params):
    n = seq.shape[0]
    e_tot = edge_index.shape[1]

    small = jnp.concatenate([seq, node_s], axis=-1)
    small = jnp.pad(small, ((0, N_PAD - n), (0, 128 - 39)))
    esm = jnp.pad(esm_emb, ((0, N_PAD - n), (0, 0)))
    wn = params["proj_node"]["W"]
    w0 = jnp.pad(wn[:39], ((0, 128 - 39), (0, 0)))
    x = _proj_node(small, w0, esm, wn[39:], params["proj_node"]["b"])

    e_pad = ((e_tot + 1023) // 1024) * 1024
    edge_sp = jnp.pad(edge_s, ((0, e_pad - e_tot), (0, 128 - edge_s.shape[1])))
    wep = jnp.pad(params["proj_edge"]["W"], ((0, 128 - edge_s.shape[1]), (0, 0)))
    eatab = _mm(edge_sp, wep, params["proj_edge"]["b"])

    src = edge_index[0].astype(jnp.int32)
    dst = edge_index[1].astype(jnp.int32)
    src_s, dst_s, perm, meta = _build_edge_meta(src, dst, N_PAD, e_tot)

    eaw = 128
    for lp in params["layers"]:
        din = lp["Wq"].shape[0]
        dout = lp["Wq"].shape[1]
        wq_ext = jnp.concatenate([lp["Wq"], lp["bq"][None, :]], axis=0)
        wq_ext = jnp.pad(wq_ext, ((0, 7), (0, 0)))
        wtl = _mm(wq_ext, lp["We"].T, jnp.zeros((eaw,), jnp.float32),
                  blk=din + 8)
        wqq = jnp.concatenate([lp["Wq"], wtl[:din]], axis=1)
        bqq = jnp.concatenate([lp["bq"], wtl[din]], axis=0)
        qqt = _mm(x, wqq, bqq)
        ktab = _mm(x, lp["Wk"], lp["bk"])
        vtab = _mm(x, lp["Wv"], lp["bv"])
        E = 48 if dout > 128 else 80
        sweep = _make_sweep(N_PAD, dout, eaw, E)
        R = sweep(ktab, qqt, vtab, eatab, src_s, dst_s, perm, meta)
        R = R.reshape(N_PAD, dout + eaw + 16)
        wcat = jnp.concatenate([lp["We"], lp["Ws"]], axis=0)
        x = _epilogue(R, x, wcat, lp["bs"], dout, eaw)

    batch_pad = jnp.pad(batch.astype(jnp.int32), (0, N_PAD - n),
                        constant_values=N_GRAPHS)
    ohT = (batch_pad[None, :]
           == jnp.arange(N_GRAPHS, dtype=jnp.int32)[:, None]).astype(jnp.float32)
    return _pool(x, ohT)


# speedup vs baseline: 3.7442x; 1.0631x over previous
import functools

import jax
import jax.numpy as jnp
import numpy as np
from jax import lax
from jax.experimental import pallas as pl
from jax.experimental.pallas import tpu as pltpu
from jax.experimental.pallas import tpu_sc as plsc

N_GRAPHS = 16
NC, NS = 2, 16
NW = NC * NS
NOLAYOUT = pltpu.CompilerParams(needs_layout_passes=False)
N_PAD = 10240
EPS = 1e-16


def _mm_kernel(a_ref, b_ref, bias_ref, o_ref):
    o_ref[...] = (jnp.dot(a_ref[...], b_ref[...],
                          preferred_element_type=jnp.float32) + bias_ref[...])


def _mm(A, B, bias, blk=1024):
    M, K = A.shape
    N = B.shape[1]
    return pl.pallas_call(
        _mm_kernel,
        grid=(M // blk,),
        in_specs=[
            pl.BlockSpec((blk, K), lambda i: (i, 0)),
            pl.BlockSpec((K, N), lambda i: (0, 0)),
            pl.BlockSpec((1, N), lambda i: (0, 0)),
        ],
        out_specs=pl.BlockSpec((blk, N), lambda i: (i, 0)),
        out_shape=jax.ShapeDtypeStruct((M, N), jnp.float32),
    )(A, B, bias.reshape(1, N))


def _proj_node_kernel(small_ref, w0_ref, esm_ref, w2_ref, bias_ref, o_ref):
    k = pl.program_id(1)

    @pl.when(k == 0)
    def _init():
        o_ref[...] = (jnp.dot(small_ref[...], w0_ref[...],
                              preferred_element_type=jnp.float32) + bias_ref[...])

    @pl.when(k > 0)
    def _acc():
        o_ref[...] += jnp.dot(esm_ref[...], w2_ref[...],
                              preferred_element_type=jnp.float32)


def _proj_node(small, w0, esm, w2, bias, blk=1024, kblk=512):
    M = small.shape[0]
    N = w0.shape[1]
    nk = esm.shape[1] // kblk
    return pl.pallas_call(
        _proj_node_kernel,
        grid=(M // blk, nk + 1),
        in_specs=[
            pl.BlockSpec((blk, small.shape[1]), lambda i, k: (i, 0)),
            pl.BlockSpec((small.shape[1], N), lambda i, k: (0, 0)),
            pl.BlockSpec((blk, kblk), lambda i, k: (i, jnp.maximum(k - 1, 0))),
            pl.BlockSpec((kblk, N), lambda i, k: (jnp.maximum(k - 1, 0), 0)),
            pl.BlockSpec((1, N), lambda i, k: (0, 0)),
        ],
        out_specs=pl.BlockSpec((blk, N), lambda i, k: (i, 0)),
        out_shape=jax.ShapeDtypeStruct((M, N), jnp.float32),
    )(small, w0, esm, w2, bias.reshape(1, N))


def _epilogue_kernel(dout, eaw, r_ref, x_ref, wcat_ref, bias_ref, o_ref):
    r = r_ref[...]
    den = lax.slice(r, (0, dout + eaw), (r.shape[0], dout + eaw + 1))
    recip = 1.0 / (den + EPS)
    rv = lax.slice(r, (0, 0), (r.shape[0], dout)) * recip
    ru = lax.slice(r, (0, dout), (r.shape[0], dout + eaw)) * recip
    a = jnp.concatenate([ru, x_ref[...]], axis=1)
    out = rv + jnp.dot(a, wcat_ref[...],
                       preferred_element_type=jnp.float32) + bias_ref[...]
    o_ref[...] = jnp.where(out >= 0, out, 0.01 * out)


def _epilogue(R, x, Wcat, bias, dout, eaw, blk=1024):
    M, W = R.shape
    kdim = Wcat.shape[0]
    return pl.pallas_call(
        functools.partial(_epilogue_kernel, dout, eaw),
        grid=(M // blk,),
        in_specs=[
            pl.BlockSpec((blk, W), lambda i: (i, 0)),
            pl.BlockSpec((blk, x.shape[1]), lambda i: (i, 0)),
            pl.BlockSpec((kdim, dout), lambda i: (0, 0)),
            pl.BlockSpec((1, dout), lambda i: (0, 0)),
        ],
        out_specs=pl.BlockSpec((blk, dout), lambda i: (i, 0)),
        out_shape=jax.ShapeDtypeStruct((M, dout), jnp.float32),
    )(R, x, Wcat, bias.reshape(1, dout))


def _pool_kernel(ohT_ref, x_ref, sum_ref, cnt_ref):
    k = pl.program_id(0)
    oh = ohT_ref[...]
    acc = jnp.dot(oh, x_ref[...], preferred_element_type=jnp.float32)
    c = jnp.sum(oh, axis=1, keepdims=True) + jnp.zeros_like(cnt_ref)

    @pl.when(k == 0)
    def _init():
        sum_ref[...] = acc
        cnt_ref[...] = c

    @pl.when(k > 0)
    def _acc():
        sum_ref[...] += acc
        cnt_ref[...] += c

    @pl.when(k == pl.num_programs(0) - 1)
    def _fin():
        sum_ref[...] = sum_ref[...] / jnp.maximum(cnt_ref[...], 1.0)


def _pool(x, ohT, blk=1024):
    M, C = x.shape
    out, _ = pl.pallas_call(
        _pool_kernel,
        grid=(M // blk,),
        in_specs=[
            pl.BlockSpec((N_GRAPHS, blk), lambda k: (0, k)),
            pl.BlockSpec((blk, C), lambda k: (k, 0)),
        ],
        out_specs=[
            pl.BlockSpec((N_GRAPHS, C), lambda k: (0, 0)),
            pl.BlockSpec((N_GRAPHS, C), lambda k: (0, 0)),
        ],
        out_shape=[
            jax.ShapeDtypeStruct((N_GRAPHS, C), jnp.float32),
            jax.ShapeDtypeStruct((N_GRAPHS, C), jnp.float32),
        ],
    )(ohT, x)
    return out


def _make_sweep(n_nodes, dout, eaw, E):
    W = dout + eaw + 16
    rsqrt_c = 1.0 / float(np.sqrt(np.float32(dout)))
    mesh = plsc.VectorSubcoreMesh(core_axis_name="c", subcore_axis_name="s")

    @functools.partial(
        pl.kernel,
        out_type=jax.ShapeDtypeStruct((n_nodes, 1, W), jnp.float32),
        mesh=mesh,
        compiler_params=NOLAYOUT,
        scratch_types=[
            pltpu.VMEM((16,), jnp.int32),
            pltpu.VMEM((3, E), jnp.int32),
            pltpu.VMEM((3, E), jnp.int32),
            pltpu.VMEM((3, E), jnp.int32),
            pltpu.VMEM((2, E, dout), jnp.float32),
            pltpu.VMEM((2, E, dout + eaw), jnp.float32),
            pltpu.VMEM((2, E, dout), jnp.float32),
            pltpu.VMEM((2, E, eaw), jnp.float32),
            pltpu.VMEM((16, 1, W), jnp.float32),
            pltpu.VMEM((16, 16), jnp.float32),
            pltpu.VMEM((E,), jnp.float32),
            pltpu.SMEM((8,), jnp.int32),
            pltpu.SemaphoreType.DMA,
            pltpu.SemaphoreType.DMA,
            pltpu.SemaphoreType.DMA,
            pltpu.SemaphoreType.DMA,
            pltpu.SemaphoreType.DMA,
            pltpu.SemaphoreType.DMA,
            pltpu.SemaphoreType.DMA,
        ],
    )
    def sweep(ktab, qqt, vtab, eatab, srcp, dstp, permp, meta, outR,
              metav, srcb, dstb, permb, kb, qb, vb, eab, win, accm, exb, st,
              s1, s2, s3, s4, si1, si2, si3):
        cid = lax.axis_index("c")
        sid = lax.axis_index("s")
        w = sid * NC + cid
        pltpu.sync_copy(meta.at[pl.ds(pl.multiple_of(w * 16, 16), 16)], metav)
        mv = metav[...]
        ts0, ts1, f0, f1 = mv[0], mv[1], mv[2], mv[3]
        cbase0 = pl.multiple_of((ts0 // 8) * 8, 8)
        nch = lax.div(ts1 - cbase0 + E - 1, E)
        st[0] = f0
        st[1] = 0
        st[2] = f0
        for r in range(16):
            for j in range(W // 16):
                win[r, 0, pl.ds(j * 16, 16)] = jnp.zeros((16,), jnp.float32)

        def adv(i, carry):
            wi = st[1] + 1

            @pl.when(wi == 16)
            def _fl():
                fb = st[2]
                pltpu.sync_copy(win, outR.at[pl.ds(fb, 16)])
                for r in range(16):
                    for j in range(W // 16):
                        win[r, 0, pl.ds(j * 16, 16)] = jnp.zeros((16,), jnp.float32)
                st[2] = fb + 16
                st[1] = 0

            @pl.when(wi < 16)
            def _nf():
                st[1] = wi

            return carry

        pltpu.sync_copy(srcp.at[pl.ds(cbase0, E)], srcb.at[0])
        pltpu.sync_copy(dstp.at[pl.ds(cbase0, E)], dstb.at[0])
        pltpu.sync_copy(permp.at[pl.ds(cbase0, E)], permb.at[0])

        def piter(k, carry):
            p3 = lax.rem(k, 3)
            p2 = lax.rem(k, 2)
            c1 = pltpu.async_copy(ktab.at[srcb.at[p3]], kb.at[p2], s1)
            c2 = pltpu.async_copy(qqt.at[dstb.at[p3]], qb.at[p2], s2)
            c3 = pltpu.async_copy(vtab.at[srcb.at[p3]], vb.at[p2], s3)
            c4 = pltpu.async_copy(eatab.at[permb.at[p3]], eab.at[p2], s4)
            n3 = lax.rem(k + 1, 3)
            cbn = pl.multiple_of(cbase0 + (k + 1) * E, 8)
            i1 = pltpu.async_copy(srcp.at[pl.ds(cbn, E)], srcb.at[n3], si1)
            i2 = pltpu.async_copy(dstp.at[pl.ds(cbn, E)], dstb.at[n3], si2)
            i3 = pltpu.async_copy(permp.at[pl.ds(cbn, E)], permb.at[n3], si3)

            @pl.when(k > 0)
            def _compute():
                kc = k - 1
                q3 = lax.rem(kc, 3)
                q2 = lax.rem(kc, 2)
                cb = pl.multiple_of(cbase0 + kc * E, 8)

                def agroup(g, carry3):
                    def alane(l, carry4):
                        el = g * 16 + l
                        acc = jnp.zeros((16,), jnp.float32)
                        for j in range(dout // 16):
                            acc = acc + (kb[q2, el, pl.ds(j * 16, 16)]
                                         * qb[q2, el, pl.ds(j * 16, 16)])
                        for j in range(eaw // 16):
                            acc = acc + (eab[q2, el, pl.ds(j * 16, 16)]
                                         * qb[q2, el, pl.ds(dout + j * 16, 16)])
                        accm[l, :] = acc
                        return carry4

                    lax.fori_loop(0, 16, alane, 0)
                    lanes = lax.iota(jnp.int32, 16)
                    al = jnp.zeros((16,), jnp.float32)
                    for c in range(16):
                        al = al + plsc.load_gather(
                            accm, [lanes, jnp.full((16,), c, jnp.int32)])
                    exb[pl.ds(pl.multiple_of(g * 16, 16), 16)] = (
                        jnp.exp(al * rsqrt_c))
                    return carry3

                lax.fori_loop(0, E // 16, agroup, 0)

                def edge(el, carry2):
                    d16 = plsc.load_gather(
                        dstb, [jnp.full((16,), q3, jnp.int32),
                               jnp.full((16,), el, jnp.int32)])
                    d = d16[0]
                    lax.fori_loop(0, d - st[0], adv, 0)
                    st[0] = d
                    ex = plsc.load_gather(exb, [jnp.full((16,), el, jnp.int32)])
                    wi = st[1]
                    for j in range(dout // 16):
                        win[wi, 0, pl.ds(j * 16, 16)] += (
                            ex * vb[q2, el, pl.ds(j * 16, 16)])
                    for j in range(eaw // 16):
                        win[wi, 0, pl.ds(dout + j * 16, 16)] += (
                            ex * eab[q2, el, pl.ds(j * 16, 16)])
                    win[wi, 0, pl.ds(dout + eaw, 16)] += ex
                    return carry2

                estart = jnp.maximum(ts0 - cb, 0)
                eend = jnp.minimum(ts1 - cb, E)
                lax.fori_loop(estart, eend, edge, 0)

            c1.wait()
            c2.wait()
            c3.wait()
            c4.wait()
            i1.wait()
            i2.wait()
            i3.wait()
            return carry

        lax.fori_loop(0, nch + 1, piter, 0)
        lax.fori_loop(0, f1 - st[0], adv, 0)

        def prow(r, carry):
            pltpu.sync_copy(win.at[r], outR.at[st[2] + r])
            return carry

        lax.fori_loop(0, st[1], prow, 0)

    return sweep


def _build_edge_meta(src, dst, n_nodes_pad, e_tot):
    perm = jnp.argsort(dst)
    dst_s = dst[perm].astype(jnp.int32)
    src_s = src[perm].astype(jnp.int32)
    perm = perm.astype(jnp.int32)
    probes = jnp.arange(NW, dtype=jnp.int32) * (e_tot // NW)
    ts_w = jnp.searchsorted(dst_s, dst_s[probes], side="left").astype(jnp.int32)
    ts = jnp.concatenate([ts_w, jnp.array([e_tot], jnp.int32)])
    f_w = dst_s[ts_w]
    f_w = f_w.at[0].set(0)
    f = jnp.concatenate([f_w, jnp.array([n_nodes_pad], jnp.int32)])
    meta = jnp.zeros((NW, 16), jnp.int32)
    meta = meta.at[:, 0].set(ts[:NW])
    meta = meta.at[:, 1].set(ts[1:])
    meta = meta.at[:, 2].set(f[:NW])
    meta = meta.at[:, 3].set(f[1:])
    src_s = jnp.pad(src_s, (0, 512))
    dst_s = jnp.pad(dst_s, (0, 512))
    perm = jnp.pad(perm, (0, 512))
    return src_s, dst_s, perm, meta.reshape(-1)


def kernel(seq, edge_index, node_s, esm_emb, edge_s, batch, params):
    n = seq.shape[0]
    e_tot = edge_index.shape[1]

    small = jnp.concatenate([seq, node_s], axis=-1)
    small = jnp.pad(small, ((0, N_PAD - n), (0, 128 - 39)))
    esm = jnp.pad(esm_emb, ((0, N_PAD - n), (0, 0)))
    wn = params["proj_node"]["W"]
    w0 = jnp.pad(wn[:39], ((0, 128 - 39), (0, 0)))
    x = _proj_node(small, w0, esm, wn[39:], params["proj_node"]["b"])

    e_pad = ((e_tot + 1023) // 1024) * 1024
    edge_sp = jnp.pad(edge_s, ((0, e_pad - e_tot), (0, 128 - edge_s.shape[1])))
    wep = jnp.pad(params["proj_edge"]["W"], ((0, 128 - edge_s.shape[1]), (0, 0)))
    eatab = _mm(edge_sp, wep, params["proj_edge"]["b"])

    src = edge_index[0].astype(jnp.int32)
    dst = edge_index[1].astype(jnp.int32)
    src_s, dst_s, perm, meta = _build_edge_meta(src, dst, N_PAD, e_tot)

    eaw = 128
    for lp in params["layers"]:
        din = lp["Wq"].shape[0]
        dout = lp["Wq"].shape[1]
        wq_ext = jnp.concatenate([lp["Wq"], lp["bq"][None, :]], axis=0)
        wq_ext = jnp.pad(wq_ext, ((0, 7), (0, 0)))
        wtl = _mm(wq_ext, lp["We"].T, jnp.zeros((eaw,), jnp.float32),
                  blk=din + 8)
        wqq = jnp.concatenate([lp["Wq"], wtl[:din]], axis=1)
        bqq = jnp.concatenate([lp["bq"], wtl[din]], axis=0)
        qqt = _mm(x, wqq, bqq)
        ktab = _mm(x, lp["Wk"], lp["bk"])
        vtab = _mm(x, lp["Wv"], lp["bv"])
        E = 48 if dout > 128 else 80
        sweep = _make_sweep(N_PAD, dout, eaw, E)
        R = sweep(ktab, qqt, vtab, eatab, src_s, dst_s, perm, meta)
        R = R.reshape(N_PAD, dout + eaw + 16)
        wcat = jnp.concatenate([lp["We"], lp["Ws"]], axis=0)
        x = _epilogue(R, x, wcat, lp["bs"], dout, eaw)

    batch_pad = jnp.pad(batch.astype(jnp.int32), (0, N_PAD - n),
                        constant_values=N_GRAPHS)
    ohT = (batch_pad[None, :]
           == jnp.arange(N_GRAPHS, dtype=jnp.int32)[:, None]).astype(jnp.float32)
    return _pool(x, ohT)


# speedup vs baseline: 3.7608x; 1.0044x over previous
import functools

import jax
import jax.numpy as jnp
import numpy as np
from jax import lax
from jax.experimental import pallas as pl
from jax.experimental.pallas import tpu as pltpu
from jax.experimental.pallas import tpu_sc as plsc

N_GRAPHS = 16
NC, NS = 2, 16
NW = NC * NS
NOLAYOUT = pltpu.CompilerParams(needs_layout_passes=False)
N_PAD = 10240
EPS = 1e-16


def _mm_kernel(a_ref, b_ref, bias_ref, o_ref):
    o_ref[...] = (jnp.dot(a_ref[...], b_ref[...],
                          preferred_element_type=jnp.float32) + bias_ref[...])


def _mm(A, B, bias, blk=1024):
    M, K = A.shape
    N = B.shape[1]
    return pl.pallas_call(
        _mm_kernel,
        grid=(M // blk,),
        in_specs=[
            pl.BlockSpec((blk, K), lambda i: (i, 0)),
            pl.BlockSpec((K, N), lambda i: (0, 0)),
            pl.BlockSpec((1, N), lambda i: (0, 0)),
        ],
        out_specs=pl.BlockSpec((blk, N), lambda i: (i, 0)),
        out_shape=jax.ShapeDtypeStruct((M, N), jnp.float32),
    )(A, B, bias.reshape(1, N))


def _proj_node_kernel(small_ref, w0_ref, esm_ref, w2_ref, bias_ref, o_ref):
    k = pl.program_id(1)

    @pl.when(k == 0)
    def _init():
        o_ref[...] = (jnp.dot(small_ref[...], w0_ref[...],
                              preferred_element_type=jnp.float32) + bias_ref[...])

    @pl.when(k > 0)
    def _acc():
        o_ref[...] += jnp.dot(esm_ref[...], w2_ref[...],
                              preferred_element_type=jnp.float32)


def _proj_node(small, w0, esm, w2, bias, blk=1024, kblk=512):
    M = small.shape[0]
    N = w0.shape[1]
    nk = esm.shape[1] // kblk
    return pl.pallas_call(
        _proj_node_kernel,
        grid=(M // blk, nk + 1),
        in_specs=[
            pl.BlockSpec((blk, small.shape[1]), lambda i, k: (i, 0)),
            pl.BlockSpec((small.shape[1], N), lambda i, k: (0, 0)),
            pl.BlockSpec((blk, kblk), lambda i, k: (i, jnp.maximum(k - 1, 0))),
            pl.BlockSpec((kblk, N), lambda i, k: (jnp.maximum(k - 1, 0), 0)),
            pl.BlockSpec((1, N), lambda i, k: (0, 0)),
        ],
        out_specs=pl.BlockSpec((blk, N), lambda i, k: (i, 0)),
        out_shape=jax.ShapeDtypeStruct((M, N), jnp.float32),
    )(small, w0, esm, w2, bias.reshape(1, N))


def _epilogue_kernel(dout, eaw, r_ref, x_ref, wcat_ref, bias_ref, o_ref):
    r = r_ref[...]
    den = lax.slice(r, (0, dout + eaw), (r.shape[0], dout + eaw + 1))
    recip = 1.0 / (den + EPS)
    rv = lax.slice(r, (0, 0), (r.shape[0], dout)) * recip
    ru = lax.slice(r, (0, dout), (r.shape[0], dout + eaw)) * recip
    a = jnp.concatenate([ru, x_ref[...]], axis=1)
    out = rv + jnp.dot(a, wcat_ref[...],
                       preferred_element_type=jnp.float32) + bias_ref[...]
    o_ref[...] = jnp.where(out >= 0, out, 0.01 * out)


def _epilogue(R, x, Wcat, bias, dout, eaw, blk=1024):
    M, W = R.shape
    kdim = Wcat.shape[0]
    return pl.pallas_call(
        functools.partial(_epilogue_kernel, dout, eaw),
        grid=(M // blk,),
        in_specs=[
            pl.BlockSpec((blk, W), lambda i: (i, 0)),
            pl.BlockSpec((blk, x.shape[1]), lambda i: (i, 0)),
            pl.BlockSpec((kdim, dout), lambda i: (0, 0)),
            pl.BlockSpec((1, dout), lambda i: (0, 0)),
        ],
        out_specs=pl.BlockSpec((blk, dout), lambda i: (i, 0)),
        out_shape=jax.ShapeDtypeStruct((M, dout), jnp.float32),
    )(R, x, Wcat, bias.reshape(1, dout))


def _pool_kernel(ohT_ref, x_ref, sum_ref, cnt_ref):
    k = pl.program_id(0)
    oh = ohT_ref[...]
    acc = jnp.dot(oh, x_ref[...], preferred_element_type=jnp.float32)
    c = jnp.sum(oh, axis=1, keepdims=True) + jnp.zeros_like(cnt_ref)

    @pl.when(k == 0)
    def _init():
        sum_ref[...] = acc
        cnt_ref[...] = c

    @pl.when(k > 0)
    def _acc():
        sum_ref[...] += acc
        cnt_ref[...] += c

    @pl.when(k == pl.num_programs(0) - 1)
    def _fin():
        sum_ref[...] = sum_ref[...] / jnp.maximum(cnt_ref[...], 1.0)


def _pool(x, ohT, blk=1024):
    M, C = x.shape
    out, _ = pl.pallas_call(
        _pool_kernel,
        grid=(M // blk,),
        in_specs=[
            pl.BlockSpec((N_GRAPHS, blk), lambda k: (0, k)),
            pl.BlockSpec((blk, C), lambda k: (k, 0)),
        ],
        out_specs=[
            pl.BlockSpec((N_GRAPHS, C), lambda k: (0, 0)),
            pl.BlockSpec((N_GRAPHS, C), lambda k: (0, 0)),
        ],
        out_shape=[
            jax.ShapeDtypeStruct((N_GRAPHS, C), jnp.float32),
            jax.ShapeDtypeStruct((N_GRAPHS, C), jnp.float32),
        ],
    )(ohT, x)
    return out


def _make_sweep(n_nodes, dout, eaw, E):
    W = dout + eaw + 16
    rsqrt_c = 1.0 / float(np.sqrt(np.float32(dout)))
    mesh = plsc.VectorSubcoreMesh(core_axis_name="c", subcore_axis_name="s")

    @functools.partial(
        pl.kernel,
        out_type=jax.ShapeDtypeStruct((n_nodes, 1, W), jnp.float32),
        mesh=mesh,
        compiler_params=NOLAYOUT,
        scratch_types=[
            pltpu.VMEM((16,), jnp.int32),
            pltpu.VMEM((3, E), jnp.int32),
            pltpu.VMEM((3, E), jnp.int32),
            pltpu.VMEM((3, E), jnp.int32),
            pltpu.VMEM((2, E, dout), jnp.float32),
            pltpu.VMEM((2, E, dout + eaw), jnp.float32),
            pltpu.VMEM((2, E, dout), jnp.float32),
            pltpu.VMEM((2, E, eaw), jnp.float32),
            pltpu.VMEM((16, 1, W), jnp.float32),
            pltpu.VMEM((16, 16), jnp.float32),
            pltpu.VMEM((E,), jnp.float32),
            pltpu.SMEM((8,), jnp.int32),
            pltpu.SemaphoreType.DMA,
            pltpu.SemaphoreType.DMA,
            pltpu.SemaphoreType.DMA,
            pltpu.SemaphoreType.DMA,
            pltpu.SemaphoreType.DMA,
            pltpu.SemaphoreType.DMA,
            pltpu.SemaphoreType.DMA,
        ],
    )
    def sweep(ktab, qqt, vtab, eatab, srcp, dstp, permp, meta, outR,
              metav, srcb, dstb, permb, kb, qb, vb, eab, win, accm, exb, st,
              s1, s2, s3, s4, si1, si2, si3):
        cid = lax.axis_index("c")
        sid = lax.axis_index("s")
        w = sid * NC + cid
        pltpu.sync_copy(meta.at[pl.ds(pl.multiple_of(w * 16, 16), 16)], metav)
        mv = metav[...]
        ts0, ts1, f0, f1 = mv[0], mv[1], mv[2], mv[3]
        cbase0 = pl.multiple_of((ts0 // 8) * 8, 8)
        nch = lax.div(ts1 - cbase0 + E - 1, E)
        st[0] = f0
        st[1] = 0
        st[2] = f0
        for r in range(16):
            for j in range(W // 16):
                win[r, 0, pl.ds(j * 16, 16)] = jnp.zeros((16,), jnp.float32)

        def adv(i, carry):
            wi = st[1] + 1

            @pl.when(wi == 16)
            def _fl():
                fb = st[2]
                pltpu.sync_copy(win, outR.at[pl.ds(fb, 16)])
                for r in range(16):
                    for j in range(W // 16):
                        win[r, 0, pl.ds(j * 16, 16)] = jnp.zeros((16,), jnp.float32)
                st[2] = fb + 16
                st[1] = 0

            @pl.when(wi < 16)
            def _nf():
                st[1] = wi

            return carry

        pltpu.sync_copy(srcp.at[pl.ds(cbase0, E)], srcb.at[0])
        pltpu.sync_copy(dstp.at[pl.ds(cbase0, E)], dstb.at[0])
        pltpu.sync_copy(permp.at[pl.ds(cbase0, E)], permb.at[0])

        def piter(k, carry):
            p3 = lax.rem(k, 3)
            p2 = lax.rem(k, 2)
            c1 = pltpu.async_copy(ktab.at[srcb.at[p3]], kb.at[p2], s1)
            c2 = pltpu.async_copy(qqt.at[dstb.at[p3]], qb.at[p2], s2)
            c3 = pltpu.async_copy(vtab.at[srcb.at[p3]], vb.at[p2], s3)
            c4 = pltpu.async_copy(eatab.at[permb.at[p3]], eab.at[p2], s4)
            n3 = lax.rem(k + 1, 3)
            cbn = pl.multiple_of(cbase0 + (k + 1) * E, 8)
            i1 = pltpu.async_copy(srcp.at[pl.ds(cbn, E)], srcb.at[n3], si1)
            i2 = pltpu.async_copy(dstp.at[pl.ds(cbn, E)], dstb.at[n3], si2)
            i3 = pltpu.async_copy(permp.at[pl.ds(cbn, E)], permb.at[n3], si3)

            @pl.when(k > 0)
            def _compute():
                kc = k - 1
                q3 = lax.rem(kc, 3)
                q2 = lax.rem(kc, 2)
                cb = pl.multiple_of(cbase0 + kc * E, 8)

                def agroup(g, carry3):
                    def alane(l, carry4):
                        el = g * 16 + l
                        acc = jnp.zeros((16,), jnp.float32)
                        for j in range(dout // 16):
                            acc = acc + (kb[q2, el, pl.ds(j * 16, 16)]
                                         * qb[q2, el, pl.ds(j * 16, 16)])
                        for j in range(eaw // 16):
                            acc = acc + (eab[q2, el, pl.ds(j * 16, 16)]
                                         * qb[q2, el, pl.ds(dout + j * 16, 16)])
                        accm[l, :] = acc
                        return carry4

                    lax.fori_loop(0, 16, alane, 0)
                    lanes = lax.iota(jnp.int32, 16)
                    al = jnp.zeros((16,), jnp.float32)
                    for c in range(16):
                        al = al + plsc.load_gather(
                            accm, [lanes, jnp.full((16,), c, jnp.int32)])
                    exb[pl.ds(pl.multiple_of(g * 16, 16), 16)] = (
                        jnp.exp(al * rsqrt_c))
                    return carry3

                lax.fori_loop(0, E // 16, agroup, 0)

                def edge(el, carry2):
                    d16 = plsc.load_gather(
                        dstb, [jnp.full((16,), q3, jnp.int32),
                               jnp.full((16,), el, jnp.int32)])
                    d = d16[0]

                    @pl.when(d != st[0])
                    def _newseg():
                        lax.fori_loop(0, d - st[0], adv, 0)
                        st[0] = d

                    ex = plsc.load_gather(exb, [jnp.full((16,), el, jnp.int32)])
                    wi = st[1]
                    for j in range(dout // 16):
                        win[wi, 0, pl.ds(j * 16, 16)] += (
                            ex * vb[q2, el, pl.ds(j * 16, 16)])
                    for j in range(eaw // 16):
                        win[wi, 0, pl.ds(dout + j * 16, 16)] += (
                            ex * eab[q2, el, pl.ds(j * 16, 16)])
                    win[wi, 0, pl.ds(dout + eaw, 16)] += ex
                    return carry2

                estart = jnp.maximum(ts0 - cb, 0)
                eend = jnp.minimum(ts1 - cb, E)
                lax.fori_loop(estart, eend, edge, 0)

            c1.wait()
            c2.wait()
            c3.wait()
            c4.wait()
            i1.wait()
            i2.wait()
            i3.wait()
            return carry

        lax.fori_loop(0, nch + 1, piter, 0)
        lax.fori_loop(0, f1 - st[0], adv, 0)

        def prow(r, carry):
            pltpu.sync_copy(win.at[r], outR.at[st[2] + r])
            return carry

        lax.fori_loop(0, st[1], prow, 0)

    return sweep


def _build_edge_meta(src, dst, n_nodes_pad, e_tot):
    perm = jnp.argsort(dst)
    dst_s = dst[perm].astype(jnp.int32)
    src_s = src[perm].astype(jnp.int32)
    perm = perm.astype(jnp.int32)
    probes = jnp.arange(NW, dtype=jnp.int32) * (e_tot // NW)
    ts_w = jnp.searchsorted(dst_s, dst_s[probes], side="left").astype(jnp.int32)
    ts = jnp.concatenate([ts_w, jnp.array([e_tot], jnp.int32)])
    f_w = dst_s[ts_w]
    f_w = f_w.at[0].set(0)
    f = jnp.concatenate([f_w, jnp.array([n_nodes_pad], jnp.int32)])
    meta = jnp.zeros((NW, 16), jnp.int32)
    meta = meta.at[:, 0].set(ts[:NW])
    meta = meta.at[:, 1].set(ts[1:])
    meta = meta.at[:, 2].set(f[:NW])
    meta = meta.at[:, 3].set(f[1:])
    src_s = jnp.pad(src_s, (0, 512))
    dst_s = jnp.pad(dst_s, (0, 512))
    perm = jnp.pad(perm, (0, 512))
    return src_s, dst_s, perm, meta.reshape(-1)


def kernel(seq, edge_index, node_s, esm_emb, edge_s, batch, params):
    n = seq.shape[0]
    e_tot = edge_index.shape[1]

    small = jnp.concatenate([seq, node_s], axis=-1)
    small = jnp.pad(small, ((0, N_PAD - n), (0, 128 - 39)))
    esm = jnp.pad(esm_emb, ((0, N_PAD - n), (0, 0)))
    wn = params["proj_node"]["W"]
    w0 = jnp.pad(wn[:39], ((0, 128 - 39), (0, 0)))
    x = _proj_node(small, w0, esm, wn[39:], params["proj_node"]["b"])

    e_pad = ((e_tot + 1023) // 1024) * 1024
    edge_sp = jnp.pad(edge_s, ((0, e_pad - e_tot), (0, 128 - edge_s.shape[1])))
    wep = jnp.pad(params["proj_edge"]["W"], ((0, 128 - edge_s.shape[1]), (0, 0)))
    eatab = _mm(edge_sp, wep, params["proj_edge"]["b"])

    src = edge_index[0].astype(jnp.int32)
    dst = edge_index[1].astype(jnp.int32)
    src_s, dst_s, perm, meta = _build_edge_meta(src, dst, N_PAD, e_tot)

    eaw = 128
    for lp in params["layers"]:
        din = lp["Wq"].shape[0]
        dout = lp["Wq"].shape[1]
        wq_ext = jnp.concatenate([lp["Wq"], lp["bq"][None, :]], axis=0)
        wq_ext = jnp.pad(wq_ext, ((0, 7), (0, 0)))
        wtl = _mm(wq_ext, lp["We"].T, jnp.zeros((eaw,), jnp.float32),
                  blk=din + 8)
        wqq = jnp.concatenate([lp["Wq"], wtl[:din]], axis=1)
        bqq = jnp.concatenate([lp["bq"], wtl[din]], axis=0)
        qqt = _mm(x, wqq, bqq)
        ktab = _mm(x, lp["Wk"], lp["bk"])
        vtab = _mm(x, lp["Wv"], lp["bv"])
        E = 48 if dout > 128 else 80
        sweep = _make_sweep(N_PAD, dout, eaw, E)
        R = sweep(ktab, qqt, vtab, eatab, src_s, dst_s, perm, meta)
        R = R.reshape(N_PAD, dout + eaw + 16)
        wcat = jnp.concatenate([lp["We"], lp["Ws"]], axis=0)
        x = _epilogue(R, x, wcat, lp["bs"], dout, eaw)

    batch_pad = jnp.pad(batch.astype(jnp.int32), (0, N_PAD - n),
                        constant_values=N_GRAPHS)
    ohT = (batch_pad[None, :]
           == jnp.arange(N_GRAPHS, dtype=jnp.int32)[:, None]).astype(jnp.float32)
    return _pool(x, ohT)


# speedup vs baseline: 3.7808x; 1.0053x over previous
import functools

import jax
import jax.numpy as jnp
import numpy as np
from jax import lax
from jax.experimental import pallas as pl
from jax.experimental.pallas import tpu as pltpu
from jax.experimental.pallas import tpu_sc as plsc

N_GRAPHS = 16
NC, NS = 2, 16
NW = NC * NS
NOLAYOUT = pltpu.CompilerParams(needs_layout_passes=False)
N_PAD = 10240
EPS = 1e-16


def _mm_kernel(a_ref, b_ref, bias_ref, o_ref):
    o_ref[...] = (jnp.dot(a_ref[...], b_ref[...],
                          preferred_element_type=jnp.float32) + bias_ref[...])


def _mm(A, B, bias, blk=1024):
    M, K = A.shape
    N = B.shape[1]
    return pl.pallas_call(
        _mm_kernel,
        grid=(M // blk,),
        in_specs=[
            pl.BlockSpec((blk, K), lambda i: (i, 0)),
            pl.BlockSpec((K, N), lambda i: (0, 0)),
            pl.BlockSpec((1, N), lambda i: (0, 0)),
        ],
        out_specs=pl.BlockSpec((blk, N), lambda i: (i, 0)),
        out_shape=jax.ShapeDtypeStruct((M, N), jnp.float32),
    )(A, B, bias.reshape(1, N))


def _proj_node_kernel(small_ref, w0_ref, esm_ref, w2_ref, bias_ref, o_ref):
    k = pl.program_id(1)

    @pl.when(k == 0)
    def _init():
        o_ref[...] = (jnp.dot(small_ref[...], w0_ref[...],
                              preferred_element_type=jnp.float32) + bias_ref[...])

    @pl.when(k > 0)
    def _acc():
        o_ref[...] += jnp.dot(esm_ref[...], w2_ref[...],
                              preferred_element_type=jnp.float32)


def _proj_node(small, w0, esm, w2, bias, blk=1024, kblk=512):
    M = small.shape[0]
    N = w0.shape[1]
    nk = esm.shape[1] // kblk
    return pl.pallas_call(
        _proj_node_kernel,
        grid=(M // blk, nk + 1),
        in_specs=[
            pl.BlockSpec((blk, small.shape[1]), lambda i, k: (i, 0)),
            pl.BlockSpec((small.shape[1], N), lambda i, k: (0, 0)),
            pl.BlockSpec((blk, kblk), lambda i, k: (i, jnp.maximum(k - 1, 0))),
            pl.BlockSpec((kblk, N), lambda i, k: (jnp.maximum(k - 1, 0), 0)),
            pl.BlockSpec((1, N), lambda i, k: (0, 0)),
        ],
        out_specs=pl.BlockSpec((blk, N), lambda i, k: (i, 0)),
        out_shape=jax.ShapeDtypeStruct((M, N), jnp.float32),
    )(small, w0, esm, w2, bias.reshape(1, N))


def _epilogue_kernel(dout, eaw, r_ref, x_ref, wcat_ref, bias_ref, o_ref):
    r = r_ref[...]
    den = lax.slice(r, (0, dout + eaw), (r.shape[0], dout + eaw + 1))
    recip = 1.0 / (den + EPS)
    rv = lax.slice(r, (0, 0), (r.shape[0], dout)) * recip
    ru = lax.slice(r, (0, dout), (r.shape[0], dout + eaw)) * recip
    a = jnp.concatenate([ru, x_ref[...]], axis=1)
    out = rv + jnp.dot(a, wcat_ref[...],
                       preferred_element_type=jnp.float32) + bias_ref[...]
    o_ref[...] = jnp.where(out >= 0, out, 0.01 * out)


def _epilogue(R, x, Wcat, bias, dout, eaw, blk=1024):
    M, W = R.shape
    kdim = Wcat.shape[0]
    return pl.pallas_call(
        functools.partial(_epilogue_kernel, dout, eaw),
        grid=(M // blk,),
        in_specs=[
            pl.BlockSpec((blk, W), lambda i: (i, 0)),
            pl.BlockSpec((blk, x.shape[1]), lambda i: (i, 0)),
            pl.BlockSpec((kdim, dout), lambda i: (0, 0)),
            pl.BlockSpec((1, dout), lambda i: (0, 0)),
        ],
        out_specs=pl.BlockSpec((blk, dout), lambda i: (i, 0)),
        out_shape=jax.ShapeDtypeStruct((M, dout), jnp.float32),
    )(R, x, Wcat, bias.reshape(1, dout))


def _pool_kernel(ohT_ref, x_ref, sum_ref, cnt_ref):
    k = pl.program_id(0)
    oh = ohT_ref[...]
    acc = jnp.dot(oh, x_ref[...], preferred_element_type=jnp.float32)
    c = jnp.sum(oh, axis=1, keepdims=True) + jnp.zeros_like(cnt_ref)

    @pl.when(k == 0)
    def _init():
        sum_ref[...] = acc
        cnt_ref[...] = c

    @pl.when(k > 0)
    def _acc():
        sum_ref[...] += acc
        cnt_ref[...] += c

    @pl.when(k == pl.num_programs(0) - 1)
    def _fin():
        sum_ref[...] = sum_ref[...] / jnp.maximum(cnt_ref[...], 1.0)


def _pool(x, ohT, blk=1024):
    M, C = x.shape
    out, _ = pl.pallas_call(
        _pool_kernel,
        grid=(M // blk,),
        in_specs=[
            pl.BlockSpec((N_GRAPHS, blk), lambda k: (0, k)),
            pl.BlockSpec((blk, C), lambda k: (k, 0)),
        ],
        out_specs=[
            pl.BlockSpec((N_GRAPHS, C), lambda k: (0, 0)),
            pl.BlockSpec((N_GRAPHS, C), lambda k: (0, 0)),
        ],
        out_shape=[
            jax.ShapeDtypeStruct((N_GRAPHS, C), jnp.float32),
            jax.ShapeDtypeStruct((N_GRAPHS, C), jnp.float32),
        ],
    )(ohT, x)
    return out


def _make_sweep(n_nodes, dout, eaw, E):
    W = dout + eaw + 16
    rsqrt_c = 1.0 / float(np.sqrt(np.float32(dout)))
    mesh = plsc.VectorSubcoreMesh(core_axis_name="c", subcore_axis_name="s")

    @functools.partial(
        pl.kernel,
        out_type=jax.ShapeDtypeStruct((n_nodes, 1, W), jnp.float32),
        mesh=mesh,
        compiler_params=NOLAYOUT,
        scratch_types=[
            pltpu.VMEM((16,), jnp.int32),
            pltpu.VMEM((3, E), jnp.int32),
            pltpu.VMEM((3, E), jnp.int32),
            pltpu.VMEM((3, E), jnp.int32),
            pltpu.VMEM((2, E, 2 * dout), jnp.float32),
            pltpu.VMEM((2, E, dout + eaw), jnp.float32),
            pltpu.VMEM((2, E, eaw), jnp.float32),
            pltpu.VMEM((16, 1, W), jnp.float32),
            pltpu.VMEM((16, 16), jnp.float32),
            pltpu.VMEM((E,), jnp.float32),
            pltpu.SMEM((8,), jnp.int32),
            pltpu.SemaphoreType.DMA,
            pltpu.SemaphoreType.DMA,
            pltpu.SemaphoreType.DMA,
            pltpu.SemaphoreType.DMA,
            pltpu.SemaphoreType.DMA,
            pltpu.SemaphoreType.DMA,
        ],
    )
    def sweep(kvtab, qqt, eatab, srcp, dstp, permp, meta, outR,
              metav, srcb, dstb, permb, kvb, qb, eab, win, accm, exb, st,
              s1, s2, s4, si1, si2, si3):
        cid = lax.axis_index("c")
        sid = lax.axis_index("s")
        w = sid * NC + cid
        pltpu.sync_copy(meta.at[pl.ds(pl.multiple_of(w * 16, 16), 16)], metav)
        mv = metav[...]
        ts0, ts1, f0, f1 = mv[0], mv[1], mv[2], mv[3]
        cbase0 = pl.multiple_of((ts0 // 8) * 8, 8)
        nch = lax.div(ts1 - cbase0 + E - 1, E)
        st[0] = f0
        st[1] = 0
        st[2] = f0
        for r in range(16):
            for j in range(W // 16):
                win[r, 0, pl.ds(j * 16, 16)] = jnp.zeros((16,), jnp.float32)

        def adv(i, carry):
            wi = st[1] + 1

            @pl.when(wi == 16)
            def _fl():
                fb = st[2]
                pltpu.sync_copy(win, outR.at[pl.ds(fb, 16)])
                for r in range(16):
                    for j in range(W // 16):
                        win[r, 0, pl.ds(j * 16, 16)] = jnp.zeros((16,), jnp.float32)
                st[2] = fb + 16
                st[1] = 0

            @pl.when(wi < 16)
            def _nf():
                st[1] = wi

            return carry

        pltpu.sync_copy(srcp.at[pl.ds(cbase0, E)], srcb.at[0])
        pltpu.sync_copy(dstp.at[pl.ds(cbase0, E)], dstb.at[0])
        pltpu.sync_copy(permp.at[pl.ds(cbase0, E)], permb.at[0])

        def piter(k, carry):
            p3 = lax.rem(k, 3)
            p2 = lax.rem(k, 2)
            c1 = pltpu.async_copy(kvtab.at[srcb.at[p3]], kvb.at[p2], s1)
            c2 = pltpu.async_copy(qqt.at[dstb.at[p3]], qb.at[p2], s2)
            c4 = pltpu.async_copy(eatab.at[permb.at[p3]], eab.at[p2], s4)
            n3 = lax.rem(k + 1, 3)
            cbn = pl.multiple_of(cbase0 + (k + 1) * E, 8)
            i1 = pltpu.async_copy(srcp.at[pl.ds(cbn, E)], srcb.at[n3], si1)
            i2 = pltpu.async_copy(dstp.at[pl.ds(cbn, E)], dstb.at[n3], si2)
            i3 = pltpu.async_copy(permp.at[pl.ds(cbn, E)], permb.at[n3], si3)

            @pl.when(k > 0)
            def _compute():
                kc = k - 1
                q3 = lax.rem(kc, 3)
                q2 = lax.rem(kc, 2)
                cb = pl.multiple_of(cbase0 + kc * E, 8)

                def agroup(g, carry3):
                    def alane(l, carry4):
                        el = g * 16 + l
                        acc = jnp.zeros((16,), jnp.float32)
                        for j in range(dout // 16):
                            acc = acc + (kvb[q2, el, pl.ds(j * 16, 16)]
                                         * qb[q2, el, pl.ds(j * 16, 16)])
                        for j in range(eaw // 16):
                            acc = acc + (eab[q2, el, pl.ds(j * 16, 16)]
                                         * qb[q2, el, pl.ds(dout + j * 16, 16)])
                        accm[l, :] = acc
                        return carry4

                    lax.fori_loop(0, 16, alane, 0)
                    lanes = lax.iota(jnp.int32, 16)
                    al = jnp.zeros((16,), jnp.float32)
                    for c in range(16):
                        al = al + plsc.load_gather(
                            accm, [lanes, jnp.full((16,), c, jnp.int32)])
                    exb[pl.ds(pl.multiple_of(g * 16, 16), 16)] = (
                        jnp.exp(al * rsqrt_c))
                    return carry3

                lax.fori_loop(0, E // 16, agroup, 0)

                def edge(el, carry2):
                    d16 = plsc.load_gather(
                        dstb, [jnp.full((16,), q3, jnp.int32),
                               jnp.full((16,), el, jnp.int32)])
                    d = d16[0]

                    @pl.when(d != st[0])
                    def _newseg():
                        lax.fori_loop(0, d - st[0], adv, 0)
                        st[0] = d

                    ex = plsc.load_gather(exb, [jnp.full((16,), el, jnp.int32)])
                    wi = st[1]
                    for j in range(dout // 16):
                        win[wi, 0, pl.ds(j * 16, 16)] += (
                            ex * kvb[q2, el, pl.ds(dout + j * 16, 16)])
                    for j in range(eaw // 16):
                        win[wi, 0, pl.ds(dout + j * 16, 16)] += (
                            ex * eab[q2, el, pl.ds(j * 16, 16)])
                    win[wi, 0, pl.ds(dout + eaw, 16)] += ex
                    return carry2

                estart = jnp.maximum(ts0 - cb, 0)
                eend = jnp.minimum(ts1 - cb, E)
                lax.fori_loop(estart, eend, edge, 0)

            c1.wait()
            c2.wait()
            c4.wait()
            i1.wait()
            i2.wait()
            i3.wait()
            return carry

        lax.fori_loop(0, nch + 1, piter, 0)
        lax.fori_loop(0, f1 - st[0], adv, 0)

        def prow(r, carry):
            pltpu.sync_copy(win.at[r], outR.at[st[2] + r])
            return carry

        lax.fori_loop(0, st[1], prow, 0)

    return sweep


def _build_edge_meta(src, dst, n_nodes_pad, e_tot):
    perm = jnp.argsort(dst)
    dst_s = dst[perm].astype(jnp.int32)
    src_s = src[perm].astype(jnp.int32)
    perm = perm.astype(jnp.int32)
    probes = jnp.arange(NW, dtype=jnp.int32) * (e_tot // NW)
    ts_w = jnp.searchsorted(dst_s, dst_s[probes], side="left").astype(jnp.int32)
    ts = jnp.concatenate([ts_w, jnp.array([e_tot], jnp.int32)])
    f_w = dst_s[ts_w]
    f_w = f_w.at[0].set(0)
    f = jnp.concatenate([f_w, jnp.array([n_nodes_pad], jnp.int32)])
    meta = jnp.zeros((NW, 16), jnp.int32)
    meta = meta.at[:, 0].set(ts[:NW])
    meta = meta.at[:, 1].set(ts[1:])
    meta = meta.at[:, 2].set(f[:NW])
    meta = meta.at[:, 3].set(f[1:])
    src_s = jnp.pad(src_s, (0, 512))
    dst_s = jnp.pad(dst_s, (0, 512))
    perm = jnp.pad(perm, (0, 512))
    return src_s, dst_s, perm, meta.reshape(-1)


def kernel(seq, edge_index, node_s, esm_emb, edge_s, batch, params):
    n = seq.shape[0]
    e_tot = edge_index.shape[1]

    small = jnp.concatenate([seq, node_s], axis=-1)
    small = jnp.pad(small, ((0, N_PAD - n), (0, 128 - 39)))
    esm = jnp.pad(esm_emb, ((0, N_PAD - n), (0, 0)))
    wn = params["proj_node"]["W"]
    w0 = jnp.pad(wn[:39], ((0, 128 - 39), (0, 0)))
    x = _proj_node(small, w0, esm, wn[39:], params["proj_node"]["b"])

    e_pad = ((e_tot + 1023) // 1024) * 1024
    edge_sp = jnp.pad(edge_s, ((0, e_pad - e_tot), (0, 128 - edge_s.shape[1])))
    wep = jnp.pad(params["proj_edge"]["W"], ((0, 128 - edge_s.shape[1]), (0, 0)))
    eatab = _mm(edge_sp, wep, params["proj_edge"]["b"])

    src = edge_index[0].astype(jnp.int32)
    dst = edge_index[1].astype(jnp.int32)
    src_s, dst_s, perm, meta = _build_edge_meta(src, dst, N_PAD, e_tot)

    eaw = 128
    for lp in params["layers"]:
        din = lp["Wq"].shape[0]
        dout = lp["Wq"].shape[1]
        wq_ext = jnp.concatenate([lp["Wq"], lp["bq"][None, :]], axis=0)
        wq_ext = jnp.pad(wq_ext, ((0, 7), (0, 0)))
        wtl = _mm(wq_ext, lp["We"].T, jnp.zeros((eaw,), jnp.float32),
                  blk=din + 8)
        wqq = jnp.concatenate([lp["Wq"], wtl[:din]], axis=1)
        bqq = jnp.concatenate([lp["bq"], wtl[din]], axis=0)
        qqt = _mm(x, wqq, bqq)
        wkv = jnp.concatenate([lp["Wk"], lp["Wv"]], axis=1)
        bkv = jnp.concatenate([lp["bk"], lp["bv"]], axis=0)
        kvtab = _mm(x, wkv, bkv)
        E = 48 if dout > 128 else 80
        sweep = _make_sweep(N_PAD, dout, eaw, E)
        R = sweep(kvtab, qqt, eatab, src_s, dst_s, perm, meta)
        R = R.reshape(N_PAD, dout + eaw + 16)
        wcat = jnp.concatenate([lp["We"], lp["Ws"]], axis=0)
        x = _epilogue(R, x, wcat, lp["bs"], dout, eaw)

    batch_pad = jnp.pad(batch.astype(jnp.int32), (0, N_PAD - n),
                        constant_values=N_GRAPHS)
    ohT = (batch_pad[None, :]
           == jnp.arange(N_GRAPHS, dtype=jnp.int32)[:, None]).astype(jnp.float32)
    return _pool(x, ohT)
